# Initial kernel scaffold; baseline (speedup 1.0000x reference)
import jax
import jax.numpy as jnp
from jax import lax
from jax.experimental import pallas as pl
from jax.experimental.pallas import tpu as pltpu
from jax.experimental.pallas import tpu_sc as plsc

_B = 4096
_L = 200
_HIDDEN = 32
_N = _B * _L
_NW = 32
_PER_W = _N // _NW
_CHUNK = 1024
_NCHUNK = _PER_W // _CHUNK


def _gather_body(idx_hbm, table_hbm, out_hbm, idx_v, rows_v, sem):
    wid = lax.axis_index("s") * 2 + lax.axis_index("c")
    base = wid * _PER_W

    def body(c, carry):
        row0 = pl.multiple_of(base + c * _CHUNK, _CHUNK)
        pltpu.sync_copy(idx_hbm.at[pl.ds(row0, _CHUNK)], idx_v)
        pltpu.async_copy(table_hbm.at[idx_v], rows_v, sem).wait()
        pltpu.sync_copy(rows_v, out_hbm.at[pl.ds(row0, _CHUNK)])
        return carry

    lax.fori_loop(0, _NCHUNK, body, 0)


def kernel(input_ids, table):
    idx = input_ids.reshape(_N).astype(jnp.int32)
    mesh = plsc.VectorSubcoreMesh(core_axis_name="c", subcore_axis_name="s")
    f = pl.kernel(
        _gather_body,
        mesh=mesh,
        out_type=jax.ShapeDtypeStruct((_N, _HIDDEN), jnp.float32),
        scratch_types=[
            pltpu.VMEM((_CHUNK,), jnp.int32),
            pltpu.VMEM((_CHUNK, _HIDDEN), jnp.float32),
            pltpu.SemaphoreType.DMA,
        ],
    )
    out = f(idx, table)
    return out.reshape(_B, _L, _HIDDEN)


# speedup vs baseline: 1.4591x; 1.4591x over previous
import jax
import jax.numpy as jnp
from jax import lax
from jax.experimental import pallas as pl
from jax.experimental.pallas import tpu as pltpu
from jax.experimental.pallas import tpu_sc as plsc

_B = 4096
_L = 200
_HIDDEN = 32
_N = _B * _L
_NW = 32
_PER_W = _N // _NW
_CHUNK = 1024
_NCHUNK = _PER_W // _CHUNK


def _gather_body(idx_hbm, table_hbm, out_hbm, idx_v, rows_v, sem):
    wid = lax.axis_index("s") * 2 + lax.axis_index("c")
    base = wid * _PER_W

    def body(c, carry):
        row0 = pl.multiple_of(base + c * _CHUNK, _CHUNK)
        pltpu.sync_copy(idx_hbm.at[pl.ds(row0, _CHUNK)], idx_v)
        pltpu.async_copy(table_hbm.at[idx_v], rows_v, sem).wait()
        pltpu.sync_copy(rows_v, out_hbm.at[pl.ds(row0, _CHUNK)])
        return carry

    lax.fori_loop(0, _NCHUNK, body, 0)


def kernel(input_ids, table):
    idx = input_ids.reshape(_N).astype(jnp.int32)
    mesh = plsc.VectorSubcoreMesh(core_axis_name="c", subcore_axis_name="s")
    f = pl.kernel(
        _gather_body,
        mesh=mesh,
        compiler_params=pltpu.CompilerParams(use_tc_tiling_on_sc=False),
        out_type=jax.ShapeDtypeStruct((_N, _HIDDEN), jnp.float32),
        scratch_types=[
            pltpu.VMEM((_CHUNK,), jnp.int32),
            pltpu.VMEM((_CHUNK, _HIDDEN), jnp.float32),
            pltpu.SemaphoreType.DMA,
        ],
    )
    out = f(idx, table)
    return out.reshape(_B, _L, _HIDDEN)


# speedup vs baseline: 1.4990x; 1.0273x over previous
import jax
import jax.numpy as jnp
from jax import lax
from jax.experimental import pallas as pl
from jax.experimental.pallas import tpu as pltpu
from jax.experimental.pallas import tpu_sc as plsc

_B = 4096
_L = 200
_HIDDEN = 32
_N = _B * _L
_NW = 32
_PER_W = _N // _NW
_C = 800
_G = 2
_NBUF = 2 * _G
_NCHUNK = _PER_W // _C
_STEPS = _NCHUNK // _G
_HALF = _STEPS // 2


def _gather_body(idx_hbm, table_hbm, out_hbm, idx_all, rows,
                 sg0, sg1, sg2, sg3, so0, so1, so2, so3, si):
    sem_g = (sg0, sg1, sg2, sg3)
    sem_o = (so0, so1, so2, so3)
    wid = lax.axis_index("s") * 2 + lax.axis_index("c")
    base = wid * _PER_W

    pltpu.async_copy(
        idx_hbm.at[pl.ds(pl.multiple_of(base, _PER_W), _PER_W)],
        idx_all, si).wait()

    def fire_gather(c, b):
        off = pl.multiple_of(c * _C, _C)
        pltpu.async_copy(
            table_hbm.at[idx_all.at[pl.ds(off, _C)]],
            rows.at[b], sem_g[b])

    def fire_wb(c, b):
        row0 = pl.multiple_of(base + c * _C, _C)
        pltpu.async_copy(rows.at[b], out_hbm.at[pl.ds(row0, _C)], sem_o[b])

    def wait_g(c, b):
        off = pl.multiple_of(c * _C, _C)
        pltpu.make_async_copy(
            table_hbm.at[idx_all.at[pl.ds(off, _C)]],
            rows.at[b], sem_g[b]).wait()

    def wait_o(c, b):
        row0 = pl.multiple_of(base + c * _C, _C)
        pltpu.make_async_copy(
            rows.at[b], out_hbm.at[pl.ds(row0, _C)], sem_o[b]).wait()

    for j in range(_G):
        fire_gather(j, j)

    def body(t, carry):
        s0 = 2 * t
        for j in range(_G):
            b = _G + j

            @pl.when(t > 0)
            def _():
                wait_o((s0 - 1) * _G + j, b)
            fire_gather((s0 + 1) * _G + j, b)
        for j in range(_G):
            wait_g(s0 * _G + j, j)
            fire_wb(s0 * _G + j, j)
        for j in range(_G):

            @pl.when(t < _HALF - 1)
            def _():
                wait_o(s0 * _G + j, j)
                fire_gather((s0 + 2) * _G + j, j)
        for j in range(_G):
            b = _G + j
            wait_g((s0 + 1) * _G + j, b)
            fire_wb((s0 + 1) * _G + j, b)
        return carry

    lax.fori_loop(0, _HALF, body, 0)

    for j in range(_G):
        wait_o((_STEPS - 2) * _G + j, j)
        wait_o((_STEPS - 1) * _G + j, _G + j)


def kernel(input_ids, table):
    idx = input_ids.reshape(_N).astype(jnp.int32)
    mesh = plsc.VectorSubcoreMesh(core_axis_name="c", subcore_axis_name="s")
    f = pl.kernel(
        _gather_body,
        mesh=mesh,
        compiler_params=pltpu.CompilerParams(use_tc_tiling_on_sc=False),
        out_type=jax.ShapeDtypeStruct((_N, _HIDDEN), jnp.float32),
        scratch_types=[
            pltpu.VMEM((_PER_W,), jnp.int32),
            pltpu.VMEM((_NBUF, _C, _HIDDEN), jnp.float32),
        ] + [pltpu.SemaphoreType.DMA] * 9,
    )
    out = f(idx, table)
    return out.reshape(_B, _L, _HIDDEN)
